# baseline (speedup 1.0000x reference)
import functools

import jax
import jax.numpy as jnp
from jax import lax
from jax.experimental import pallas as pl
from jax.experimental.pallas import tpu as pltpu
from jax.experimental.pallas import tpu_sc as plsc

NC = 2
NS = 16
CHUNK = 128
NBUF = 4
LAG = 2


def kernel(x, edge_index, W_self, b_self, W_neigh, b_neigh):
    n_nodes, feat = x.shape
    out_dim = W_self.shape[0]
    n_edges = edge_index.shape[1]
    hf = feat // 2

    wave = NS * CHUNK * 8
    edges_pad = -(-n_edges // wave) * wave
    rows = edges_pad // CHUNK
    rpt = rows // NS

    sh_rows = -(-(n_nodes + 1) // (NS * 8)) * NS * 8
    zrows = sh_rows // NS

    ei = edge_index.astype(jnp.int32)
    pad = edges_pad - n_edges
    src_flat = jnp.concatenate([ei[0], jnp.zeros((pad,), jnp.int32)])
    dst_p = jnp.concatenate([ei[1], jnp.full((pad,), n_nodes, jnp.int32)]
                            ).reshape(rows, CHUNK)
    src2 = jnp.stack([src_flat, src_flat + n_nodes]).reshape(NC, rows, CHUNK)
    xs = jnp.concatenate([x, x], axis=0)
    z_agg = jnp.zeros((zrows, feat), jnp.float32)
    z_deg = jnp.zeros((zrows, 16), jnp.float32)

    mesh = plsc.VectorSubcoreMesh(core_axis_name="c", subcore_axis_name="s")

    @functools.partial(
        pl.kernel,
        mesh=mesh,
        compiler_params=pltpu.CompilerParams(use_tc_tiling_on_sc=False),
        out_type=(
            jax.ShapeDtypeStruct((NC, sh_rows, feat), jnp.float32),
        ),
        scratch_types=[
            pltpu.VMEM((rpt, CHUNK), jnp.int32),
            pltpu.VMEM((NBUF, CHUNK, feat), jnp.float32),
            pltpu.VMEM_SHARED((sh_rows // 2, feat), jnp.float32),
            pltpu.SemaphoreType.DMA,
        ],
    )
    def sc_agg(xs_hbm, src_hbm, dst_hbm, za_hbm, zd_hbm,
               agg_out,
               src_v, rows_v, agg_sh, sem_g):
        c = lax.axis_index("c")
        s = lax.axis_index("s")

        r0 = s * zrows
        pltpu.sync_copy(za_hbm, agg_sh.at[pl.ds(0, zrows)])

        base = s * rpt
        pltpu.sync_copy(src_hbm.at[c, pl.ds(base, rpt)], src_v)
        plsc.subcore_barrier()

        half = rpt // 2
        off = c * half

        for b in range(LAG):
            pltpu.async_copy(xs_hbm.at[src_v.at[off + b]], rows_v.at[b], sem_g)

        def body(g, _):
            for b in range(NBUF):
                j = g * NBUF + b
                pltpu.make_async_copy(xs_hbm.at[src_v.at[off + j]], rows_v.at[b], sem_g).wait()

                nxt = j + LAG
                bn = (b + LAG) % NBUF

                @pl.when(nxt < half)
                def _():
                    pltpu.async_copy(xs_hbm.at[src_v.at[off + nxt]], rows_v.at[bn], sem_g)
            return 0
        lax.fori_loop(0, half // NBUF, body, 0)
        plsc.subcore_barrier()

        pltpu.sync_copy(agg_sh.at[pl.ds(0, zrows)], agg_out.at[c, pl.ds(r0, zrows)])

    agg_p, = sc_agg(xs, src2, dst_p, z_agg, z_deg)
    deg_p = jnp.zeros((NC, sh_rows, 16), jnp.float32)

    bm = 1000
    grid = n_nodes // bm

    def tc_body(x_ref, a_ref, d_ref, ws_ref, wl_ref, wh_ref, bs_ref, bn_ref, o_ref):
        d = d_ref[0, :, 0:1] + d_ref[1, :, 0:1]
        h = jnp.dot(x_ref[...], ws_ref[...], preferred_element_type=jnp.float32)
        h = h + jnp.dot(a_ref[0][:, :64], wl_ref[...], preferred_element_type=jnp.float32)
        h = h + jnp.dot(a_ref[1][:, 64:], wh_ref[...], preferred_element_type=jnp.float32)
        h = h + bs_ref[...] + d * bn_ref[...]
        o_ref[...] = jnp.maximum(h, 0.0)

    out = pl.pallas_call(
        tc_body,
        grid=(grid,),
        in_specs=[
            pl.BlockSpec((bm, feat), lambda i: (i, 0)),
            pl.BlockSpec((NC, bm, feat), lambda i: (0, i, 0)),
            pl.BlockSpec((NC, bm, 16), lambda i: (0, i, 0)),
            pl.BlockSpec((feat, out_dim), lambda i: (0, 0)),
            pl.BlockSpec((hf, out_dim), lambda i: (0, 0)),
            pl.BlockSpec((hf, out_dim), lambda i: (0, 0)),
            pl.BlockSpec((1, out_dim), lambda i: (0, 0)),
            pl.BlockSpec((1, out_dim), lambda i: (0, 0)),
        ],
        out_specs=pl.BlockSpec((bm, out_dim), lambda i: (i, 0)),
        out_shape=jax.ShapeDtypeStruct((n_nodes, out_dim), jnp.float32),
    )(x, agg_p, deg_p, W_self.T, W_neigh[:, :hf].T, W_neigh[:, hf:].T,
      b_self.reshape(1, -1), b_neigh.reshape(1, -1))
    return out


# speedup vs baseline: 1.4020x; 1.4020x over previous
import functools

import jax
import jax.numpy as jnp
from jax import lax
from jax.experimental import pallas as pl
from jax.experimental.pallas import tpu as pltpu
from jax.experimental.pallas import tpu_sc as plsc

NC = 2
NS = 16
CHUNK = 128
NBUF = 4
LAG = 2


def kernel(x, edge_index, W_self, b_self, W_neigh, b_neigh):
    n_nodes, feat = x.shape
    out_dim = W_self.shape[0]
    n_edges = edge_index.shape[1]
    hf = feat // 2

    wave = NS * CHUNK * 8
    edges_pad = -(-n_edges // wave) * wave
    rows = edges_pad // CHUNK
    rpt = rows // NS

    sh_rows = -(-(n_nodes + 1) // (NS * 8)) * NS * 8
    zrows = sh_rows // NS

    ei = edge_index.astype(jnp.int32)
    pad = edges_pad - n_edges
    src_flat = jnp.concatenate([ei[0], jnp.zeros((pad,), jnp.int32)])
    dst_p = jnp.concatenate([ei[1], jnp.full((pad,), n_nodes, jnp.int32)]
                            ).reshape(rows, CHUNK)
    src2 = jnp.stack([src_flat, src_flat + n_nodes]).reshape(NC, rows, CHUNK)
    xs = jnp.concatenate([x[:, :hf], x[:, hf:]], axis=0)
    z_agg = jnp.zeros((zrows, hf), jnp.float32)
    z_deg = jnp.zeros((zrows, 16), jnp.float32)

    mesh = plsc.VectorSubcoreMesh(core_axis_name="c", subcore_axis_name="s")

    @functools.partial(
        pl.kernel,
        mesh=mesh,
        compiler_params=pltpu.CompilerParams(use_tc_tiling_on_sc=False),
        out_type=(
            jax.ShapeDtypeStruct((NC, sh_rows, hf), jnp.float32),
            jax.ShapeDtypeStruct((NC, sh_rows, 16), jnp.float32),
        ),
        scratch_types=[
            pltpu.VMEM((rpt, CHUNK), jnp.int32),
            pltpu.VMEM((rpt, CHUNK), jnp.int32),
            pltpu.VMEM((NBUF, CHUNK, hf), jnp.float32),
            pltpu.VMEM((CHUNK, 16), jnp.float32),
            pltpu.VMEM_SHARED((sh_rows, hf), jnp.float32),
            pltpu.VMEM_SHARED((sh_rows, 16), jnp.float32),
            pltpu.SemaphoreType.DMA,
            pltpu.SemaphoreType.DMA,
        ],
    )
    def sc_agg(xs_hbm, src_hbm, dst_hbm, za_hbm, zd_hbm,
               agg_out, deg_out,
               src_v, dst_v, rows_v, ones_v, agg_sh, deg_sh, sem_g, sem_s):
        c = lax.axis_index("c")
        s = lax.axis_index("s")

        def fill_ones(i, _):
            ones_v[i, :] = jnp.ones((16,), jnp.float32)
            return 0
        lax.fori_loop(0, CHUNK, fill_ones, 0)

        r0 = s * zrows
        pltpu.sync_copy(za_hbm, agg_sh.at[pl.ds(r0, zrows)])
        pltpu.sync_copy(zd_hbm, deg_sh.at[pl.ds(r0, zrows)])

        base = s * rpt
        pltpu.sync_copy(src_hbm.at[c, pl.ds(base, rpt)], src_v)
        pltpu.sync_copy(dst_hbm.at[pl.ds(base, rpt)], dst_v)
        plsc.subcore_barrier()

        half = rpt // 2

        def counted(j):
            return jnp.where(c == 0, j < half, j >= half)

        for b in range(LAG):
            pltpu.async_copy(xs_hbm.at[src_v.at[b]], rows_v.at[b], sem_g)

        def body(g, _):
            for b in range(NBUF):
                j = g * NBUF + b
                pltpu.make_async_copy(xs_hbm.at[src_v.at[j]], rows_v.at[b], sem_g).wait()
                pltpu.async_copy(rows_v.at[b], agg_sh.at[dst_v.at[j]], sem_s, add=True)

                @pl.when(counted(j))
                def _():
                    pltpu.async_copy(ones_v, deg_sh.at[dst_v.at[j]], sem_s, add=True)

                jl = j - LAG
                bl = (b - LAG) % NBUF

                @pl.when(jl >= 0)
                def _():
                    pltpu.make_async_copy(rows_v.at[bl], agg_sh.at[dst_v.at[jl]], sem_s).wait()

                @pl.when((jl >= 0) & counted(jl))
                def _():
                    pltpu.make_async_copy(ones_v, deg_sh.at[dst_v.at[jl]], sem_s).wait()

                nxt = j + LAG
                bn = (b + LAG) % NBUF

                @pl.when(nxt < rpt)
                def _():
                    pltpu.async_copy(xs_hbm.at[src_v.at[nxt]], rows_v.at[bn], sem_g)
            return 0
        lax.fori_loop(0, rpt // NBUF, body, 0)

        for k in range(rpt - LAG, rpt):
            pltpu.make_async_copy(rows_v.at[k % NBUF], agg_sh.at[dst_v.at[k]], sem_s).wait()

            @pl.when(counted(k))
            def _():
                pltpu.make_async_copy(ones_v, deg_sh.at[dst_v.at[k]], sem_s).wait()
        plsc.subcore_barrier()

        pltpu.sync_copy(agg_sh.at[pl.ds(r0, zrows)], agg_out.at[c, pl.ds(r0, zrows)])
        pltpu.sync_copy(deg_sh.at[pl.ds(r0, zrows)], deg_out.at[c, pl.ds(r0, zrows)])

    agg_p, deg_p = sc_agg(xs, src2, dst_p, z_agg, z_deg)

    bm = 1000
    grid = n_nodes // bm

    def tc_body(x_ref, a_ref, d_ref, ws_ref, wl_ref, wh_ref, bs_ref, bn_ref, o_ref):
        d = d_ref[0, :, 0:1] + d_ref[1, :, 0:1]
        h = jnp.dot(x_ref[...], ws_ref[...], preferred_element_type=jnp.float32)
        h = h + jnp.dot(a_ref[0], wl_ref[...], preferred_element_type=jnp.float32)
        h = h + jnp.dot(a_ref[1], wh_ref[...], preferred_element_type=jnp.float32)
        h = h + bs_ref[...] + d * bn_ref[...]
        o_ref[...] = jnp.maximum(h, 0.0)

    out = pl.pallas_call(
        tc_body,
        grid=(grid,),
        in_specs=[
            pl.BlockSpec((bm, feat), lambda i: (i, 0)),
            pl.BlockSpec((NC, bm, hf), lambda i: (0, i, 0)),
            pl.BlockSpec((NC, bm, 16), lambda i: (0, i, 0)),
            pl.BlockSpec((feat, out_dim), lambda i: (0, 0)),
            pl.BlockSpec((hf, out_dim), lambda i: (0, 0)),
            pl.BlockSpec((hf, out_dim), lambda i: (0, 0)),
            pl.BlockSpec((1, out_dim), lambda i: (0, 0)),
            pl.BlockSpec((1, out_dim), lambda i: (0, 0)),
        ],
        out_specs=pl.BlockSpec((bm, out_dim), lambda i: (i, 0)),
        out_shape=jax.ShapeDtypeStruct((n_nodes, out_dim), jnp.float32),
    )(x, agg_p, deg_p, W_self.T, W_neigh[:, :hf].T, W_neigh[:, hf:].T,
      b_self.reshape(1, -1), b_neigh.reshape(1, -1))
    return out


# speedup vs baseline: 1.5136x; 1.0796x over previous
import functools

import jax
import jax.numpy as jnp
from jax import lax
from jax.experimental import pallas as pl
from jax.experimental.pallas import tpu as pltpu
from jax.experimental.pallas import tpu_sc as plsc

NC = 2
NS = 16
CHUNK = 128
NBUF = 4
LAG = 2


def kernel(x, edge_index, W_self, b_self, W_neigh, b_neigh):
    n_nodes, feat = x.shape
    out_dim = W_self.shape[0]
    n_edges = edge_index.shape[1]
    hf = feat // 2

    wave = NS * CHUNK * 8
    edges_pad = -(-n_edges // wave) * wave
    rows = edges_pad // CHUNK
    rpt = rows // NS

    sh_rows = -(-(n_nodes + 1) // (NS * 8)) * NS * 8
    zrows = sh_rows // NS

    ei = edge_index.astype(jnp.int32)
    pad = edges_pad - n_edges
    src_flat = jnp.concatenate([ei[0], jnp.zeros((pad,), jnp.int32)])
    dst_p = jnp.concatenate([ei[1], jnp.full((pad,), n_nodes, jnp.int32)]
                            ).reshape(rows, CHUNK)
    src2 = jnp.stack([src_flat, src_flat + n_nodes]).reshape(NC, rows, CHUNK)
    xs = jnp.concatenate([x[:, :hf], x[:, hf:]], axis=0)
    z_agg = jnp.zeros((zrows, hf), jnp.float32)
    z_deg = jnp.zeros((zrows, 16), jnp.float32)

    mesh = plsc.VectorSubcoreMesh(core_axis_name="c", subcore_axis_name="s")

    @functools.partial(
        pl.kernel,
        mesh=mesh,
        compiler_params=pltpu.CompilerParams(use_tc_tiling_on_sc=False),
        out_type=(
            jax.ShapeDtypeStruct((NC, sh_rows, hf), jnp.float32),
            jax.ShapeDtypeStruct((NC, sh_rows, 16), jnp.float32),
        ),
        scratch_types=[
            pltpu.VMEM((rpt, CHUNK), jnp.int32),
            pltpu.VMEM((rpt, CHUNK), jnp.int32),
            pltpu.VMEM((NBUF, CHUNK, hf), jnp.float32),
            pltpu.VMEM((CHUNK, 16), jnp.float32),
            pltpu.VMEM_SHARED((sh_rows, hf), jnp.float32),
            pltpu.VMEM_SHARED((sh_rows, 16), jnp.float32),
            pltpu.SemaphoreType.DMA,
            pltpu.SemaphoreType.DMA,
        ],
    )
    def sc_agg(xs_hbm, src_hbm, dst_hbm, za_hbm, zd_hbm,
               agg_out, deg_out,
               src_v, dst_v, rows_v, ones_v, agg_sh, deg_sh, sem_g, sem_s):
        c = lax.axis_index("c")
        s = lax.axis_index("s")

        def fill_ones(i, _):
            ones_v[i, :] = jnp.ones((16,), jnp.float32)
            return 0
        lax.fori_loop(0, CHUNK, fill_ones, 0)

        r0 = s * zrows
        base = s * rpt
        pltpu.async_copy(za_hbm, agg_sh.at[pl.ds(r0, zrows)], sem_s)
        pltpu.async_copy(zd_hbm, deg_sh.at[pl.ds(r0, zrows)], sem_s)
        pltpu.async_copy(src_hbm.at[c, pl.ds(base, rpt)], src_v, sem_g)
        pltpu.async_copy(dst_hbm.at[pl.ds(base, rpt)], dst_v, sem_g)
        pltpu.make_async_copy(za_hbm, agg_sh.at[pl.ds(r0, zrows)], sem_s).wait()
        pltpu.make_async_copy(zd_hbm, deg_sh.at[pl.ds(r0, zrows)], sem_s).wait()
        pltpu.make_async_copy(src_hbm.at[c, pl.ds(base, rpt)], src_v, sem_g).wait()
        pltpu.make_async_copy(dst_hbm.at[pl.ds(base, rpt)], dst_v, sem_g).wait()
        plsc.subcore_barrier()

        half = rpt // 2

        def counted(j):
            return jnp.where(c == 0, j < half, j >= half)

        for b in range(LAG):
            pltpu.async_copy(xs_hbm.at[src_v.at[b]], rows_v.at[b], sem_g)

        def body(g, _):
            for b in range(NBUF):
                j = g * NBUF + b
                pltpu.make_async_copy(xs_hbm.at[src_v.at[j]], rows_v.at[b], sem_g).wait()
                pltpu.async_copy(rows_v.at[b], agg_sh.at[dst_v.at[j]], sem_s, add=True)

                @pl.when(counted(j))
                def _():
                    pltpu.async_copy(ones_v, deg_sh.at[dst_v.at[j]], sem_s, add=True)

                jl = j - LAG
                bl = (b - LAG) % NBUF

                @pl.when(jl >= 0)
                def _():
                    pltpu.make_async_copy(rows_v.at[bl], agg_sh.at[dst_v.at[jl]], sem_s).wait()

                @pl.when((jl >= 0) & counted(jl))
                def _():
                    pltpu.make_async_copy(ones_v, deg_sh.at[dst_v.at[jl]], sem_s).wait()

                nxt = j + LAG
                bn = (b + LAG) % NBUF

                @pl.when(nxt < rpt)
                def _():
                    pltpu.async_copy(xs_hbm.at[src_v.at[nxt]], rows_v.at[bn], sem_g)
            return 0
        lax.fori_loop(0, rpt // NBUF, body, 0)

        for k in range(rpt - LAG, rpt):
            pltpu.make_async_copy(rows_v.at[k % NBUF], agg_sh.at[dst_v.at[k]], sem_s).wait()

            @pl.when(counted(k))
            def _():
                pltpu.make_async_copy(ones_v, deg_sh.at[dst_v.at[k]], sem_s).wait()
        plsc.subcore_barrier()

        pltpu.sync_copy(agg_sh.at[pl.ds(r0, zrows)], agg_out.at[c, pl.ds(r0, zrows)])
        pltpu.sync_copy(deg_sh.at[pl.ds(r0, zrows)], deg_out.at[c, pl.ds(r0, zrows)])

    agg_p, deg_p = sc_agg(xs, src2, dst_p, z_agg, z_deg)

    bm = 1000
    grid = n_nodes // bm

    def tc_body(x_ref, a_ref, d_ref, ws_ref, wl_ref, wh_ref, bs_ref, bn_ref, o_ref):
        d = d_ref[0, :, 0:1] + d_ref[1, :, 0:1]
        h = jnp.dot(x_ref[...], ws_ref[...], preferred_element_type=jnp.float32)
        h = h + jnp.dot(a_ref[0], wl_ref[...], preferred_element_type=jnp.float32)
        h = h + jnp.dot(a_ref[1], wh_ref[...], preferred_element_type=jnp.float32)
        h = h + bs_ref[...] + d * bn_ref[...]
        o_ref[...] = jnp.maximum(h, 0.0)

    out = pl.pallas_call(
        tc_body,
        grid=(grid,),
        in_specs=[
            pl.BlockSpec((bm, feat), lambda i: (i, 0)),
            pl.BlockSpec((NC, bm, hf), lambda i: (0, i, 0)),
            pl.BlockSpec((NC, bm, 16), lambda i: (0, i, 0)),
            pl.BlockSpec((feat, out_dim), lambda i: (0, 0)),
            pl.BlockSpec((hf, out_dim), lambda i: (0, 0)),
            pl.BlockSpec((hf, out_dim), lambda i: (0, 0)),
            pl.BlockSpec((1, out_dim), lambda i: (0, 0)),
            pl.BlockSpec((1, out_dim), lambda i: (0, 0)),
        ],
        out_specs=pl.BlockSpec((bm, out_dim), lambda i: (i, 0)),
        out_shape=jax.ShapeDtypeStruct((n_nodes, out_dim), jnp.float32),
    )(x, agg_p, deg_p, W_self.T, W_neigh[:, :hf].T, W_neigh[:, hf:].T,
      b_self.reshape(1, -1), b_neigh.reshape(1, -1))
    return out
